# baseline (speedup 1.0000x reference)
import functools

import jax
import jax.numpy as jnp
from jax import lax
from jax.experimental import pallas as pl
from jax.experimental.pallas import tpu as pltpu
from jax.experimental.pallas import tpu_sc as plsc

DIM = 2048
E = 8
TOPK = 2
INTER = 2048
TOK = 2048

A = TOK * TOPK
B = 128
GMAX = 40
GPAD = 48
PPAD = GMAX * B
KI = INTER // 2
BT = 512

NC, NS = 2, 16
NW = NC * NS


def _router_body(x_ref, gw_ref, eidx_ref, wts_ref, xb_ref):
    xb_ref[...] = x_ref[...].astype(jnp.bfloat16)
    s = jax.nn.sigmoid(
        lax.dot_general(x_ref[...], gw_ref[...], (((1,), (1,)), ((), ())),
                        preferred_element_type=jnp.float32))
    idx = lax.broadcasted_iota(jnp.int32, (BT, E), 1)
    m1 = jnp.max(s, axis=1, keepdims=True)
    i1 = jnp.min(jnp.where(s == m1, idx, E), axis=1, keepdims=True)
    sm = jnp.where(idx == i1, -jnp.inf, s)
    m2 = jnp.max(sm, axis=1, keepdims=True)
    i2 = jnp.min(jnp.where(sm == m2, idx, E), axis=1, keepdims=True)
    wsum = m1 + m2
    eidx_ref[...] = jnp.concatenate([i1, i2], axis=1)
    wts_ref[...] = jnp.concatenate([m1 / wsum, m2 / wsum], axis=1)


def _router(x, gate_w):
    return pl.pallas_call(
        _router_body,
        grid=(TOK // BT,),
        in_specs=[
            pl.BlockSpec((BT, DIM), lambda t: (t, 0)),
            pl.BlockSpec((E, DIM), lambda t: (0, 0)),
        ],
        out_specs=[
            pl.BlockSpec((BT, TOPK), lambda t: (t, 0)),
            pl.BlockSpec((BT, TOPK), lambda t: (t, 0)),
            pl.BlockSpec((BT, DIM), lambda t: (t, 0)),
        ],
        out_shape=[
            jax.ShapeDtypeStruct((TOK, TOPK), jnp.int32),
            jax.ShapeDtypeStruct((TOK, TOPK), jnp.float32),
            jax.ShapeDtypeStruct((TOK, DIM), jnp.bfloat16),
        ],
        compiler_params=pltpu.CompilerParams(
            dimension_semantics=("parallel",)),
    )(x, gate_w)


def _dispatch_body(eidx_hbm, wts_hbm, pos_hbm, tok_hbm, wsort_hbm, teid_hbm,
                   tval_hbm, eid_v, wts_v, pos_v, tok_v, wsort_v, meta_v):
    wid = lax.axis_index("s") * NC + lax.axis_index("c")

    @pl.when(wid == 0)
    def _():
        pltpu.sync_copy(eidx_hbm, eid_v)
        pltpu.sync_copy(wts_hbm, wts_v)

        zero16 = jnp.zeros((16,), jnp.int32)

        def zero_body(i, c):
            tok_v[pl.ds(i * 16, 16)] = zero16
            return c

        lax.fori_loop(0, PPAD // 16, zero_body, 0)

        def cnt_body(c, cnts):
            v = eid_v[pl.ds(c * 16, 16)]
            return tuple(
                cnts[e] + jnp.sum(jnp.where(v == e, 1, 0).astype(jnp.int32))
                for e in range(E))

        cnts = lax.fori_loop(0, A // 16, cnt_body,
                             tuple(jnp.int32(0) for _ in range(E)))

        ts = [jnp.int32(0)]
        for e in range(E):
            ts.append(ts[e] + (cnts[e] + (B - 1)) // B)
        total_tiles = ts[E]

        iota16 = lax.iota(jnp.int32, 16)
        for c in range(GPAD // 16):
            g16 = jnp.full((16,), c * 16, jnp.int32) + iota16
            acc = jnp.full((16,), -1, jnp.int32)
            for e in range(E):
                acc = acc + jnp.where(g16 >= jnp.full((16,), ts[e], jnp.int32),
                                      1, 0).astype(jnp.int32)
            val = jnp.where(g16 < jnp.full((16,), total_tiles, jnp.int32),
                            1, 0).astype(jnp.int32)
            meta_v[pl.ds(c * 16, 16)] = jnp.maximum(acc, 0)
            meta_v[pl.ds(GPAD + c * 16, 16)] = val

        def pos_body(c, bases):
            v = eid_v[pl.ds(c * 16, 16)]
            w = wts_v[pl.ds(c * 16, 16)]
            pos16 = jnp.zeros((16,), jnp.int32)
            newb = []
            for e in range(E):
                m = v == e
                inc = jnp.where(m, 1, 0).astype(jnp.int32)
                r = plsc.cumsum(inc)
                pos_e = jnp.full((16,), bases[e], jnp.int32) + r - 1
                pos16 = jnp.where(m, pos_e, pos16)
                newb.append(bases[e] + jnp.sum(inc))
            pos_v[pl.ds(c * 16, 16)] = pos16
            tok16 = jnp.full((16,), c * 8, jnp.int32) + (iota16 >> 1)
            plsc.store_scatter(tok_v, [pos16], tok16)
            plsc.store_scatter(wsort_v, [pos16], w)
            return tuple(newb)

        lax.fori_loop(0, A // 16, pos_body,
                      tuple(ts[e] * B for e in range(E)))

        pltpu.sync_copy(pos_v, pos_hbm)
        pltpu.sync_copy(tok_v, tok_hbm)
        pltpu.sync_copy(wsort_v, wsort_hbm)
        pltpu.sync_copy(meta_v.at[pl.ds(0, GPAD)], teid_hbm)
        pltpu.sync_copy(meta_v.at[pl.ds(GPAD, GPAD)], tval_hbm)


def _dispatch(eidx_flat, wts_flat):
    mesh = plsc.VectorSubcoreMesh(core_axis_name="c", subcore_axis_name="s")
    return pl.kernel(
        _dispatch_body,
        out_type=[
            jax.ShapeDtypeStruct((A,), jnp.int32),
            jax.ShapeDtypeStruct((PPAD,), jnp.int32),
            jax.ShapeDtypeStruct((PPAD,), jnp.float32),
            jax.ShapeDtypeStruct((GPAD,), jnp.int32),
            jax.ShapeDtypeStruct((GPAD,), jnp.int32),
        ],
        mesh=mesh,
        scratch_types=[
            pltpu.VMEM((A,), jnp.int32),
            pltpu.VMEM((A,), jnp.float32),
            pltpu.VMEM((A,), jnp.int32),
            pltpu.VMEM((PPAD,), jnp.int32),
            pltpu.VMEM((PPAD,), jnp.float32),
            pltpu.VMEM((2 * GPAD,), jnp.int32),
        ],
        compiler_params=pltpu.CompilerParams(needs_layout_passes=False),
    )(eidx_flat, wts_flat)


def _gather_xs(tok_ref, xb_ref):
    oh = (tok_ref[...] == lax.broadcasted_iota(jnp.int32, (B, TOK), 1)
          ).astype(jnp.bfloat16)
    return lax.dot_general(oh, xb_ref[...], (((1,), (0,)), ((), ())),
                           preferred_element_type=jnp.float32)


def _gmm0_body(eid_ref, val_ref, xb_ref, tok_ref, w1_ref, b1_ref, w2_ref,
               b2_ref, ws_ref, ys_ref):
    @pl.when(val_ref[pl.program_id(0)] == 1)
    def _():
        xs = _gather_xs(tok_ref, xb_ref)
        h = lax.dot_general(xs, w1_ref[0], (((1,), (1,)), ((), ())),
                            preferred_element_type=jnp.float32)
        h = jnp.maximum(h + b1_ref[0], 0.0)
        o = lax.dot_general(h, w2_ref[0], (((1,), (1,)), ((), ())),
                            preferred_element_type=jnp.float32)
        ys_ref[...] = ((o + b2_ref[0]) * ws_ref[...]).astype(jnp.bfloat16)


def _gmm1_body(eid_ref, val_ref, xb_ref, tok_ref, w1_ref, b1_ref, w2_ref,
               ws_ref, ysin_ref, ys_ref):
    @pl.when(val_ref[pl.program_id(0)] == 1)
    def _():
        xs = _gather_xs(tok_ref, xb_ref)
        h = lax.dot_general(xs, w1_ref[0], (((1,), (1,)), ((), ())),
                            preferred_element_type=jnp.float32)
        h = jnp.maximum(h + b1_ref[0], 0.0)
        o = lax.dot_general(h, w2_ref[0], (((1,), (1,)), ((), ())),
                            preferred_element_type=jnp.float32)
        ys_ref[...] = ysin_ref[...].astype(jnp.float32) + o * ws_ref[...]


def _gmm(xb, tok_col, W1, b1, W2, b2, wsort_col, tile_eid, tile_valid):
    b1r = b1.reshape(E, 1, INTER)
    ws_spec = pl.BlockSpec((B, 1), lambda g, eid, val: (g, 0))
    ys_spec = pl.BlockSpec((B, DIM), lambda g, eid, val: (g, 0))
    cp = pltpu.CompilerParams(dimension_semantics=("arbitrary",))

    def w_specs(kh):
        return [
            pl.BlockSpec((TOK, DIM), lambda g, eid, val: (0, 0)),
            pl.BlockSpec((B, 1), lambda g, eid, val: (g, 0)),
            pl.BlockSpec((1, KI, DIM), lambda g, eid, val: (eid[g], kh, 0)),
            pl.BlockSpec((1, 1, KI), lambda g, eid, val: (eid[g], 0, kh)),
            pl.BlockSpec((1, DIM, KI), lambda g, eid, val: (eid[g], 0, kh)),
        ]

    ys0 = pl.pallas_call(
        _gmm0_body,
        grid_spec=pltpu.PrefetchScalarGridSpec(
            num_scalar_prefetch=2,
            grid=(GMAX,),
            in_specs=w_specs(0) + [
                pl.BlockSpec((1, 1, DIM), lambda g, eid, val: (eid[g], 0, 0)),
                ws_spec,
            ],
            out_specs=ys_spec,
        ),
        out_shape=jax.ShapeDtypeStruct((PPAD, DIM), jnp.bfloat16),
        compiler_params=cp,
    )(tile_eid, tile_valid, xb, tok_col, W1, b1r, W2,
      b2.reshape(E, 1, DIM), wsort_col)

    return pl.pallas_call(
        _gmm1_body,
        grid_spec=pltpu.PrefetchScalarGridSpec(
            num_scalar_prefetch=2,
            grid=(GMAX,),
            in_specs=w_specs(1) + [ws_spec, ys_spec],
            out_specs=ys_spec,
        ),
        out_shape=jax.ShapeDtypeStruct((PPAD, DIM), jnp.float32),
        compiler_params=cp,
    )(tile_eid, tile_valid, xb, tok_col, W1, b1r, W2, wsort_col, ys0)


def _combine_body(ys_hbm, pos_hbm, y_hbm, pos_v, rows_a, rows_b, acc_v,
                  sem_a, sem_b):
    wid = lax.axis_index("s") * NC + lax.axis_index("c")
    tpw = TOK // NW
    t0 = wid * tpw
    pltpu.sync_copy(pos_hbm.at[pl.ds(t0 * TOPK, tpw * TOPK)], pos_v)
    nr = tpw * TOPK // 16

    def src(r):
        return ys_hbm.at[pos_v.at[pl.ds(r * 16, 16)]]

    def do_round(r, rows_v):
        for j in range(8):

            def d_body(d16, cc):
                for du in range(8):
                    sl = pl.ds(d16 * 128 + du * 16, 16)
                    acc_v[j, sl] = rows_v[2 * j, sl] + rows_v[2 * j + 1, sl]
                return cc

            lax.fori_loop(0, DIM // 128, d_body, 0)
        pltpu.sync_copy(acc_v, y_hbm.at[pl.ds(t0 + r * 8, 8)])

    pltpu.async_copy(src(0), rows_a, sem_a)

    def round_body(r2, c):
        hb = pltpu.async_copy(src(2 * r2 + 1), rows_b, sem_b)
        pltpu.make_async_copy(src(2 * r2), rows_a, sem_a).wait()
        do_round(2 * r2, rows_a)

        @pl.when(r2 < nr // 2 - 1)
        def _():
            pltpu.async_copy(src(2 * r2 + 2), rows_a, sem_a)

        hb.wait()
        do_round(2 * r2 + 1, rows_b)
        return c

    lax.fori_loop(0, nr // 2, round_body, 0)


def _combine(ys, pos_flat):
    mesh = plsc.VectorSubcoreMesh(core_axis_name="c", subcore_axis_name="s")
    return pl.kernel(
        _combine_body,
        out_type=jax.ShapeDtypeStruct((TOK, DIM), jnp.float32),
        mesh=mesh,
        scratch_types=[
            pltpu.VMEM((TOK // NW * TOPK,), jnp.int32),
            pltpu.VMEM((16, DIM), jnp.float32),
            pltpu.VMEM((16, DIM), jnp.float32),
            pltpu.VMEM((8, DIM), jnp.float32),
            pltpu.SemaphoreType.DMA,
            pltpu.SemaphoreType.DMA,
        ],
        compiler_params=pltpu.CompilerParams(needs_layout_passes=False),
    )(ys, pos_flat)


@jax.jit
def _moe(x, gate_w, W1, b1, W2, b2):
    eidx, wts, xb = _router(x, gate_w)
    pos, tok_sorted, wsort, tile_eid, tile_valid = _dispatch(
        eidx.reshape(-1), wts.reshape(-1))
    ys = _gmm(xb, tok_sorted.reshape(PPAD, 1), W1, b1, W2, b2,
              wsort.reshape(PPAD, 1), tile_eid, tile_valid)
    return _combine(ys, pos)


def kernel(x, gate_w, W1, b1, W2, b2):
    return _moe(x, gate_w, W1, b1, W2, b2)


# speedup vs baseline: 1.0802x; 1.0802x over previous
import functools

import jax
import jax.numpy as jnp
from jax import lax
from jax.experimental import pallas as pl
from jax.experimental.pallas import tpu as pltpu
from jax.experimental.pallas import tpu_sc as plsc

DIM = 2048
E = 8
TOPK = 2
INTER = 2048
TOK = 2048

A = TOK * TOPK
B = 128
GMAX = 40
GPAD = 48
PPAD = GMAX * B
KI = INTER // 2
BT = 512

NC, NS = 2, 16
NW = NC * NS


def _router_body(x_ref, gw_ref, eidx_ref, wts_ref, xb_ref):
    xb_ref[...] = x_ref[...].astype(jnp.bfloat16)
    s = jax.nn.sigmoid(
        lax.dot_general(x_ref[...], gw_ref[...], (((1,), (1,)), ((), ())),
                        preferred_element_type=jnp.float32))
    idx = lax.broadcasted_iota(jnp.int32, (BT, E), 1)
    m1 = jnp.max(s, axis=1, keepdims=True)
    i1 = jnp.min(jnp.where(s == m1, idx, E), axis=1, keepdims=True)
    sm = jnp.where(idx == i1, -jnp.inf, s)
    m2 = jnp.max(sm, axis=1, keepdims=True)
    i2 = jnp.min(jnp.where(sm == m2, idx, E), axis=1, keepdims=True)
    wsum = m1 + m2
    eidx_ref[...] = jnp.concatenate([i1, i2], axis=1)
    wts_ref[...] = jnp.concatenate([m1 / wsum, m2 / wsum], axis=1)


def _router(x, gate_w):
    return pl.pallas_call(
        _router_body,
        grid=(TOK // BT,),
        in_specs=[
            pl.BlockSpec((BT, DIM), lambda t: (t, 0)),
            pl.BlockSpec((E, DIM), lambda t: (0, 0)),
        ],
        out_specs=[
            pl.BlockSpec((BT, TOPK), lambda t: (t, 0)),
            pl.BlockSpec((BT, TOPK), lambda t: (t, 0)),
            pl.BlockSpec((BT, DIM), lambda t: (t, 0)),
        ],
        out_shape=[
            jax.ShapeDtypeStruct((TOK, TOPK), jnp.int32),
            jax.ShapeDtypeStruct((TOK, TOPK), jnp.float32),
            jax.ShapeDtypeStruct((TOK, DIM), jnp.bfloat16),
        ],
        compiler_params=pltpu.CompilerParams(
            dimension_semantics=("parallel",)),
    )(x, gate_w)


def _dispatch_body(eidx_hbm, wts_hbm, pos_hbm, tok_hbm, wsort_hbm, teid_hbm,
                   tval_hbm, eid_v, wts_v, pos_v, tok_v, wsort_v, meta_v):
    wid = lax.axis_index("s") * NC + lax.axis_index("c")

    @pl.when(wid == 0)
    def _():
        pltpu.sync_copy(eidx_hbm, eid_v)
        pltpu.sync_copy(wts_hbm, wts_v)

        zero16 = jnp.zeros((16,), jnp.int32)

        def zero_body(i, c):
            tok_v[pl.ds(i * 16, 16)] = zero16
            return c

        lax.fori_loop(0, PPAD // 16, zero_body, 0)

        def cnt_body(c, cnts):
            v = eid_v[pl.ds(c * 16, 16)]
            return tuple(
                cnts[e] + jnp.sum(jnp.where(v == e, 1, 0).astype(jnp.int32))
                for e in range(E))

        cnts = lax.fori_loop(0, A // 16, cnt_body,
                             tuple(jnp.int32(0) for _ in range(E)))

        ts = [jnp.int32(0)]
        for e in range(E):
            ts.append(ts[e] + (cnts[e] + (B - 1)) // B)
        total_tiles = ts[E]

        iota16 = lax.iota(jnp.int32, 16)
        for c in range(GPAD // 16):
            g16 = jnp.full((16,), c * 16, jnp.int32) + iota16
            acc = jnp.full((16,), -1, jnp.int32)
            for e in range(E):
                acc = acc + jnp.where(g16 >= jnp.full((16,), ts[e], jnp.int32),
                                      1, 0).astype(jnp.int32)
            val = jnp.where(g16 < jnp.full((16,), total_tiles, jnp.int32),
                            1, 0).astype(jnp.int32)
            meta_v[pl.ds(c * 16, 16)] = jnp.maximum(acc, 0)
            meta_v[pl.ds(GPAD + c * 16, 16)] = val

        def pos_body(c, bases):
            v = eid_v[pl.ds(c * 16, 16)]
            w = wts_v[pl.ds(c * 16, 16)]
            pos16 = jnp.zeros((16,), jnp.int32)
            newb = []
            for e in range(E):
                m = v == e
                inc = jnp.where(m, 1, 0).astype(jnp.int32)
                r = plsc.cumsum(inc)
                pos_e = jnp.full((16,), bases[e], jnp.int32) + r - 1
                pos16 = jnp.where(m, pos_e, pos16)
                newb.append(bases[e] + jnp.sum(inc))
            pos_v[pl.ds(c * 16, 16)] = pos16
            tok16 = jnp.full((16,), c * 8, jnp.int32) + (iota16 >> 1)
            plsc.store_scatter(tok_v, [pos16], tok16)
            plsc.store_scatter(wsort_v, [pos16], w)
            return tuple(newb)

        lax.fori_loop(0, A // 16, pos_body,
                      tuple(ts[e] * B for e in range(E)))

        pltpu.sync_copy(pos_v, pos_hbm)
        pltpu.sync_copy(tok_v, tok_hbm)
        pltpu.sync_copy(wsort_v, wsort_hbm)
        pltpu.sync_copy(meta_v.at[pl.ds(0, GPAD)], teid_hbm)
        pltpu.sync_copy(meta_v.at[pl.ds(GPAD, GPAD)], tval_hbm)


def _dispatch(eidx_flat, wts_flat):
    mesh = plsc.VectorSubcoreMesh(core_axis_name="c", subcore_axis_name="s")
    return pl.kernel(
        _dispatch_body,
        out_type=[
            jax.ShapeDtypeStruct((A,), jnp.int32),
            jax.ShapeDtypeStruct((PPAD,), jnp.int32),
            jax.ShapeDtypeStruct((PPAD,), jnp.float32),
            jax.ShapeDtypeStruct((GPAD,), jnp.int32),
            jax.ShapeDtypeStruct((GPAD,), jnp.int32),
        ],
        mesh=mesh,
        scratch_types=[
            pltpu.VMEM((A,), jnp.int32),
            pltpu.VMEM((A,), jnp.float32),
            pltpu.VMEM((A,), jnp.int32),
            pltpu.VMEM((PPAD,), jnp.int32),
            pltpu.VMEM((PPAD,), jnp.float32),
            pltpu.VMEM((2 * GPAD,), jnp.int32),
        ],
        compiler_params=pltpu.CompilerParams(needs_layout_passes=False),
    )(eidx_flat, wts_flat)


def _gather_xs(tok_ref, xb_ref):
    oh = (tok_ref[...] == lax.broadcasted_iota(jnp.int32, (B, TOK), 1)
          ).astype(jnp.bfloat16)
    return lax.dot_general(oh, xb_ref[...], (((1,), (0,)), ((), ())),
                           preferred_element_type=jnp.float32)


def _gmm0_body(eid_ref, val_ref, xb_ref, tok_ref, w1_ref, b1_ref, w2_ref,
               b2_ref, ws_ref, ys_ref, xs_ref):
    @pl.when(val_ref[pl.program_id(0)] == 1)
    def _():
        xs = _gather_xs(tok_ref, xb_ref)
        xs_ref[...] = xs.astype(jnp.bfloat16)
        h = lax.dot_general(xs, w1_ref[0], (((1,), (1,)), ((), ())),
                            preferred_element_type=jnp.float32)
        h = jnp.maximum(h + b1_ref[0], 0.0)
        o = lax.dot_general(h, w2_ref[0], (((1,), (1,)), ((), ())),
                            preferred_element_type=jnp.float32)
        ys_ref[...] = ((o + b2_ref[0]) * ws_ref[...]).astype(jnp.bfloat16)


def _gmm1_body(eid_ref, val_ref, xs_ref, w1_ref, b1_ref, w2_ref,
               ws_ref, ysin_ref, ys_ref):
    @pl.when(val_ref[pl.program_id(0)] == 1)
    def _():
        xs = xs_ref[...].astype(jnp.float32)
        h = lax.dot_general(xs, w1_ref[0], (((1,), (1,)), ((), ())),
                            preferred_element_type=jnp.float32)
        h = jnp.maximum(h + b1_ref[0], 0.0)
        o = lax.dot_general(h, w2_ref[0], (((1,), (1,)), ((), ())),
                            preferred_element_type=jnp.float32)
        ys_ref[...] = ysin_ref[...].astype(jnp.float32) + o * ws_ref[...]


def _gmm(xb, tok_col, W1, b1, W2, b2, wsort_col, tile_eid, tile_valid):
    b1r = b1.reshape(E, 1, INTER)
    ws_spec = pl.BlockSpec((B, 1), lambda g, eid, val: (g, 0))
    ys_spec = pl.BlockSpec((B, DIM), lambda g, eid, val: (g, 0))
    cp = pltpu.CompilerParams(dimension_semantics=("arbitrary",))

    def w_specs(kh):
        return [
            pl.BlockSpec((TOK, DIM), lambda g, eid, val: (0, 0)),
            pl.BlockSpec((B, 1), lambda g, eid, val: (g, 0)),
            pl.BlockSpec((1, KI, DIM), lambda g, eid, val: (eid[g], kh, 0)),
            pl.BlockSpec((1, 1, KI), lambda g, eid, val: (eid[g], 0, kh)),
            pl.BlockSpec((1, DIM, KI), lambda g, eid, val: (eid[g], 0, kh)),
        ]

    ys0, xs = pl.pallas_call(
        _gmm0_body,
        grid_spec=pltpu.PrefetchScalarGridSpec(
            num_scalar_prefetch=2,
            grid=(GMAX,),
            in_specs=w_specs(0) + [
                pl.BlockSpec((1, 1, DIM), lambda g, eid, val: (eid[g], 0, 0)),
                ws_spec,
            ],
            out_specs=[ys_spec,
                       pl.BlockSpec((B, DIM), lambda g, eid, val: (g, 0))],
        ),
        out_shape=[jax.ShapeDtypeStruct((PPAD, DIM), jnp.bfloat16),
                   jax.ShapeDtypeStruct((PPAD, DIM), jnp.bfloat16)],
        compiler_params=cp,
    )(tile_eid, tile_valid, xb, tok_col, W1, b1r, W2,
      b2.reshape(E, 1, DIM), wsort_col)

    return pl.pallas_call(
        _gmm1_body,
        grid_spec=pltpu.PrefetchScalarGridSpec(
            num_scalar_prefetch=2,
            grid=(GMAX,),
            in_specs=[
                pl.BlockSpec((B, DIM), lambda g, eid, val: (g, 0)),
                pl.BlockSpec((1, KI, DIM), lambda g, eid, val: (eid[g], 1, 0)),
                pl.BlockSpec((1, 1, KI), lambda g, eid, val: (eid[g], 0, 1)),
                pl.BlockSpec((1, DIM, KI), lambda g, eid, val: (eid[g], 0, 1)),
                ws_spec,
                ys_spec,
            ],
            out_specs=ys_spec,
        ),
        out_shape=jax.ShapeDtypeStruct((PPAD, DIM), jnp.float32),
        compiler_params=cp,
    )(tile_eid, tile_valid, xs, W1, b1r, W2, wsort_col, ys0)


def _combine_body(ys_hbm, pos_hbm, y_hbm, pos_v, rows_a, rows_b, acc_v,
                  sem_a, sem_b):
    wid = lax.axis_index("s") * NC + lax.axis_index("c")
    tpw = TOK // NW
    t0 = wid * tpw
    pltpu.sync_copy(pos_hbm.at[pl.ds(t0 * TOPK, tpw * TOPK)], pos_v)
    nr = tpw * TOPK // 16

    def src(r):
        return ys_hbm.at[pos_v.at[pl.ds(r * 16, 16)]]

    def do_round(r, rows_v):
        for j in range(8):

            def d_body(d16, cc):
                for du in range(8):
                    sl = pl.ds(d16 * 128 + du * 16, 16)
                    acc_v[j, sl] = rows_v[2 * j, sl] + rows_v[2 * j + 1, sl]
                return cc

            lax.fori_loop(0, DIM // 128, d_body, 0)
        pltpu.sync_copy(acc_v, y_hbm.at[pl.ds(t0 + r * 8, 8)])

    pltpu.async_copy(src(0), rows_a, sem_a)

    def round_body(r2, c):
        hb = pltpu.async_copy(src(2 * r2 + 1), rows_b, sem_b)
        pltpu.make_async_copy(src(2 * r2), rows_a, sem_a).wait()
        do_round(2 * r2, rows_a)

        @pl.when(r2 < nr // 2 - 1)
        def _():
            pltpu.async_copy(src(2 * r2 + 2), rows_a, sem_a)

        hb.wait()
        do_round(2 * r2 + 1, rows_b)
        return c

    lax.fori_loop(0, nr // 2, round_body, 0)


def _combine(ys, pos_flat):
    mesh = plsc.VectorSubcoreMesh(core_axis_name="c", subcore_axis_name="s")
    return pl.kernel(
        _combine_body,
        out_type=jax.ShapeDtypeStruct((TOK, DIM), jnp.float32),
        mesh=mesh,
        scratch_types=[
            pltpu.VMEM((TOK // NW * TOPK,), jnp.int32),
            pltpu.VMEM((16, DIM), jnp.float32),
            pltpu.VMEM((16, DIM), jnp.float32),
            pltpu.VMEM((8, DIM), jnp.float32),
            pltpu.SemaphoreType.DMA,
            pltpu.SemaphoreType.DMA,
        ],
        compiler_params=pltpu.CompilerParams(needs_layout_passes=False),
    )(ys, pos_flat)


@jax.jit
def _moe(x, gate_w, W1, b1, W2, b2):
    eidx, wts, xb = _router(x, gate_w)
    pos, tok_sorted, wsort, tile_eid, tile_valid = _dispatch(
        eidx.reshape(-1), wts.reshape(-1))
    ys = _gmm(xb, tok_sorted.reshape(PPAD, 1), W1, b1, W2, b2,
              wsort.reshape(PPAD, 1), tile_eid, tile_valid)
    return _combine(ys, pos)


def kernel(x, gate_w, W1, b1, W2, b2):
    return _moe(x, gate_w, W1, b1, W2, b2)
